# baseline (speedup 1.0000x reference)
import functools
import jax
import jax.numpy as jnp
from jax import lax
from jax.experimental import pallas as pl
from jax.experimental.pallas import tpu as pltpu
from jax.experimental.pallas import tpu_sc as plsc

B = 4096
D0N, D1N = 10, 20
W1, W2, W3 = 32, 16, 64
H0W = D0N + W1 + W2
H1W = D1N + W3
V = 100000
V2 = 1000
NFULL = V // 128
TAIL0 = NFULL * 128
PASS_BLKS = 7
NPASS = 4
MAXOFF = NFULL - PASS_BLKS
GRP = 256

_info = plsc.get_sparse_core_info()
_NC, _NS = _info.num_cores, _info.num_subcores
NW = _NC * _NS
BPW = B // NW
L = 16

_mesh = plsc.VectorSubcoreMesh(core_axis_name="c", subcore_axis_name="s")
_params = pltpu.CompilerParams(needs_layout_passes=False)


@functools.partial(
    pl.kernel,
    mesh=_mesh,
    out_type=(
        jax.ShapeDtypeStruct((B + 1, 128), jnp.float32),
        jax.ShapeDtypeStruct((B + 1, 128), jnp.float32),
    ),
    scratch_types=[
        pltpu.VMEM((2, B), jnp.int32),
        pltpu.VMEM((1, B), jnp.int32),
        pltpu.VMEM((32, 128), jnp.int32),
        pltpu.VMEM((32, 128), jnp.int32),
        pltpu.VMEM((32, 128), jnp.int32),
        pltpu.VMEM((32, 128), jnp.int32),
        pltpu.VMEM((64, PASS_BLKS * 128), jnp.float32),
        pltpu.VMEM((GRP, 128), jnp.float32),
        pltpu.VMEM((32, W1), jnp.float32),
        pltpu.VMEM((32, W3), jnp.float32),
        pltpu.SemaphoreType.DMA,
    ],
    compiler_params=_params,
)
def _sc_gather(x0ct, x1ct, e1t, e3t, e1tl, e3tl, g1, g3,
               cv0, cv1, hitb1, hitv1, hitb3, hitv3, sblk, stage,
               tl1, tl3, sem):
    wid = lax.axis_index("s") * _NC + lax.axis_index("c")
    lo = (wid * NFULL) // NW
    hi = ((wid + 1) * NFULL) // NW
    hi_sel = jnp.where(wid == NW - 1, NFULL + 1, hi)

    iota = lax.iota(jnp.int32, L)
    zeros = jnp.zeros((L,), jnp.int32)
    bfill = jnp.full((L,), B, jnp.int32)

    pltpu.sync_copy(x0ct, cv0)
    pltpu.sync_copy(x1ct, cv1)
    pltpu.sync_copy(e1tl, tl1)
    pltpu.sync_copy(e3tl, tl3)

    def scan_body(k, cnts):
        c1, c3 = cnts
        col = iota + k * L

        def one(cv, hitb, hitv, cnt):
            v = plsc.load_gather(cv, [zeros, col])
            blk = jax.lax.shift_right_logical(v, 7)
            m = (blk >= lo) & (blk < hi_sel)
            ind = jnp.where(m, 1, 0).astype(jnp.int32)
            pos = cnt + plsc.cumsum(ind) - 1
            prow = jax.lax.shift_right_logical(pos, 7)
            pcol = pos & 127
            plsc.store_scatter(hitb, [prow, pcol], col, mask=m)
            plsc.store_scatter(hitv, [prow, pcol], v, mask=m)
            return cnt + jnp.sum(ind)

        c1 = one(cv0, hitb1, hitv1, c1)
        c3 = one(cv1, hitb3, hitv3, c3)
        return (c1, c3)

    cnt1, cnt3 = lax.fori_loop(0, B // L, scan_body,
                               (jnp.int32(0), jnp.int32(0)))

    def extract_table(e_t, tl, g_out, hitb, hitv, count, W):
        ngroups = (count + (GRP - 1)) // GRP

        def fill(c, _):
            slot = c * L + iota
            m = slot >= count
            plsc.store_scatter(hitb, [jax.lax.shift_right_logical(slot, 7),
                                      slot & 127], bfill, mask=m)
            return _
        lax.fori_loop(count // L, ngroups * (GRP // L), fill, jnp.int32(0))

        def group(g, _):
            gbase = g * GRP

            def chunk_body(c, off, tail):
                q = 2 * g + jax.lax.shift_right_logical(c, 3)
                qv = jnp.full((L,), q, jnp.int32)
                slot = c * L + iota
                valid = (gbase + slot) < count
                vv = plsc.load_gather(hitv, [qv, slot & 127])
                blk = jax.lax.shift_right_logical(vv, 7)
                if tail:
                    mp = (blk == NFULL) & valid
                    srow = vv - TAIL0
                    for f in range(W):
                        fv = jnp.full((L,), f, jnp.int32)
                        x = plsc.load_gather(tl, [srow, fv], mask=mp)
                        plsc.store_scatter(stage, [slot, fv], x, mask=mp)
                else:
                    mp = (blk >= off) & (blk < off + PASS_BLKS) & valid
                    srow = (blk - off) * 128 + (vv & 127)
                    for f in range(W):
                        fv = jnp.full((L,), f, jnp.int32)
                        x = plsc.load_gather(sblk, [fv, srow], mask=mp)
                        plsc.store_scatter(stage, [slot, fv], x, mask=mp)

            for p in range(NPASS):
                off = jnp.minimum(lo + PASS_BLKS * p, MAXOFF)
                offc = pl.multiple_of(off * 128, 128)
                pltpu.sync_copy(e_t.at[:, pl.ds(offc, PASS_BLKS * 128)],
                                sblk.at[pl.ds(0, W), :])

                def pass_chunk(c, _c, _off=off):
                    chunk_body(c, _off, False)
                    return _c
                lax.fori_loop(0, GRP // L, pass_chunk, jnp.int32(0))

            def tail_chunk(c, _c):
                chunk_body(c, jnp.int32(0), True)
                return _c
            lax.fori_loop(0, GRP // L, tail_chunk, jnp.int32(0))

            pltpu.async_copy(stage.at[pl.ds(0, 128), :],
                             g_out.at[hitb.at[2 * g]], sem).wait()
            pltpu.async_copy(stage.at[pl.ds(128, 128), :],
                             g_out.at[hitb.at[2 * g + 1]], sem).wait()
            return _

        lax.fori_loop(0, ngroups, group, jnp.int32(0))

    extract_table(e1t, tl1, g1, hitb1, hitv1, cnt1, W1)
    extract_table(e3t, tl3, g3, hitb3, hitv3, cnt3, W3)


@functools.partial(
    pl.kernel,
    mesh=_mesh,
    out_type=(
        jax.ShapeDtypeStruct((H0W, B), jnp.float32),
        jax.ShapeDtypeStruct((H1W, B), jnp.float32),
    ),
    scratch_types=[
        pltpu.VMEM((2, BPW), jnp.int32),
        pltpu.VMEM((1, BPW), jnp.int32),
        pltpu.VMEM((D0N, BPW), jnp.float32),
        pltpu.VMEM((D1N, BPW), jnp.float32),
        pltpu.VMEM((W2, V2), jnp.float32),
        pltpu.VMEM((BPW, 128), jnp.float32),
        pltpu.VMEM((BPW, 128), jnp.float32),
        pltpu.VMEM((H0W, BPW), jnp.float32),
        pltpu.VMEM((H1W, BPW), jnp.float32),
    ],
    compiler_params=_params,
)
def _sc_concat(x0ct, x1ct, x0nt, x1nt, e2t, g1, g3, h0t, h1t,
               cv0, cv1, nb0, nb1, e2v, g1v, g3v, cbuf0, cbuf1):
    wid = lax.axis_index("s") * _NC + lax.axis_index("c")
    base = wid * BPW
    cols = pl.ds(pl.multiple_of(base, 128), BPW)
    iota = lax.iota(jnp.int32, L)
    zeros = jnp.zeros((L,), jnp.int32)
    ones = jnp.full((L,), 1, jnp.int32)

    pltpu.sync_copy(x0ct.at[:, cols], cv0)
    pltpu.sync_copy(x1ct.at[:, cols], cv1)
    pltpu.sync_copy(x0nt.at[:, cols], nb0)
    pltpu.sync_copy(x1nt.at[:, cols], nb1)
    pltpu.sync_copy(e2t, e2v)
    pltpu.sync_copy(g1.at[pl.ds(pl.multiple_of(base, 128), BPW), :], g1v)
    pltpu.sync_copy(g3.at[pl.ds(pl.multiple_of(base, 128), BPW), :], g3v)

    def chunk(k, _):
        it = iota + k * L

        for src, dst, nrows, r0 in ((nb0, cbuf0, D0N, 0),
                                    (nb1, cbuf1, D1N, 0)):
            for f in range(nrows):
                fv = jnp.full((L,), f, jnp.int32)
                x = plsc.load_gather(src, [fv, it])
                plsc.store_scatter(dst, [fv + r0, it], x)

        for src, dst, nrows, r0 in ((g1v, cbuf0, W1, D0N),
                                    (g3v, cbuf1, W3, D1N)):
            for f in range(nrows):
                fv = jnp.full((L,), f, jnp.int32)
                x = plsc.load_gather(src, [it, fv])
                plsc.store_scatter(dst, [fv + r0, it], x)

        v2 = plsc.load_gather(cv0, [ones, it])
        for f in range(W2):
            fv = jnp.full((L,), f, jnp.int32)
            x = plsc.load_gather(e2v, [fv, v2])
            plsc.store_scatter(cbuf0, [fv + (D0N + W1), it], x)
        return _

    lax.fori_loop(0, BPW // L, chunk, jnp.int32(0))

    pltpu.sync_copy(cbuf0, h0t.at[:, cols])
    pltpu.sync_copy(cbuf1, h1t.at[:, cols])


def kernel(X0_num, X0_cat, X1_num, X1_cat, X2_num, E_cat1, E_cat2, E_cat3):
    E1t, E3t, E2t = E_cat1.T, E_cat3.T, E_cat2.T
    X0ct, X1ct = X0_cat.T, X1_cat.T
    X0nt, X1nt = X0_num.T, X1_num.T
    E1tl = E_cat1[TAIL0:, :]
    E3tl = E_cat3[TAIL0:, :]
    G1, G3 = _sc_gather(X0ct, X1ct, E1t, E3t, E1tl, E3tl)
    H0t, H1t = _sc_concat(X0ct, X1ct, X0nt, X1nt, E2t, G1, G3)
    return (H0t.T, H1t.T, X2_num)


# speedup vs baseline: 2.3099x; 2.3099x over previous
import functools
import jax
import jax.numpy as jnp
from jax import lax
from jax.experimental import pallas as pl
from jax.experimental.pallas import tpu as pltpu
from jax.experimental.pallas import tpu_sc as plsc

B = 4096
D0N, D1N = 10, 20
W1, W2, W3 = 32, 16, 64
H0W = D0N + W1 + W2
H1W = D1N + W3
V = 100000
V2 = 1000
NFULL = V // 128
TAIL0 = NFULL * 128
BLKS_PER_T = 49
PASS_BLKS = 3
NPASS = 17
MAXOFF = NFULL - PASS_BLKS
GRP = 256
HB = B // 2

_info = plsc.get_sparse_core_info()
_NC, _NS = _info.num_cores, _info.num_subcores
NW = _NC * _NS
BPW = B // NW
L = 16

_mesh = plsc.VectorSubcoreMesh(core_axis_name="c", subcore_axis_name="s")
_params = pltpu.CompilerParams(needs_layout_passes=False)


@functools.partial(
    pl.kernel,
    mesh=_mesh,
    out_type=(
        jax.ShapeDtypeStruct((B, 128), jnp.float32),
        jax.ShapeDtypeStruct((B, 128), jnp.float32),
    ),
    scratch_types=[
        pltpu.VMEM((2, HB), jnp.int32),
        pltpu.VMEM((1, HB), jnp.int32),
        pltpu.VMEM((16, 128), jnp.int32),
        pltpu.VMEM((16, 128), jnp.int32),
        pltpu.VMEM((16, 128), jnp.int32),
        pltpu.VMEM((16, 128), jnp.int32),
        pltpu.VMEM((64, PASS_BLKS * 128), jnp.float32),
        pltpu.VMEM((GRP, 128), jnp.float32),
        pltpu.VMEM((32, W1), jnp.float32),
        pltpu.VMEM((32, W3), jnp.float32),
        pltpu.VMEM_SHARED((HB + 1, 128), jnp.float32),
        pltpu.VMEM_SHARED((HB + 1, 128), jnp.float32),
        pltpu.SemaphoreType.DMA,
    ],
    compiler_params=_params,
)
def _sc_gather(x0ct, x1ct, e1t, e3t, e1tl, e3tl, g1, g3,
               cv0, cv1, hitb1, hitv1, hitb3, hitv3, sblk, stage,
               tl1, tl3, gsh1, gsh3, sem):
    cid = lax.axis_index("c")
    t = lax.axis_index("s")
    lo = t * BLKS_PER_T
    hi = jnp.minimum(lo + BLKS_PER_T, NFULL)
    hi_sel = jnp.where(t == _NS - 1, NFULL + 1, hi)
    bbase = pl.multiple_of(cid * HB, 128)

    iota = lax.iota(jnp.int32, L)
    zeros = jnp.zeros((L,), jnp.int32)
    bfill = jnp.full((L,), HB, jnp.int32)

    pltpu.sync_copy(x0ct.at[:, pl.ds(bbase, HB)], cv0)
    pltpu.sync_copy(x1ct.at[:, pl.ds(bbase, HB)], cv1)
    pltpu.sync_copy(e1tl, tl1)
    pltpu.sync_copy(e3tl, tl3)

    def scan_body(k, cnts):
        c1, c3 = cnts
        col = iota + k * L

        def one(cv, hitb, hitv, cnt):
            v = plsc.load_gather(cv, [zeros, col])
            blk = jax.lax.shift_right_logical(v, 7)
            m = (blk >= lo) & (blk < hi_sel)
            ind = jnp.where(m, 1, 0).astype(jnp.int32)

            @pl.when(jnp.any(m))
            def _():
                pos = cnt + plsc.cumsum(ind) - 1
                prow = jax.lax.shift_right_logical(pos, 7)
                pcol = pos & 127
                plsc.store_scatter(hitb, [prow, pcol], col, mask=m)
                plsc.store_scatter(hitv, [prow, pcol], v, mask=m)
            return cnt + plsc.all_reduce_population_count(m)

        c1 = one(cv0, hitb1, hitv1, c1)
        c3 = one(cv1, hitb3, hitv3, c3)
        return (c1, c3)

    cnt1v, cnt3v = lax.fori_loop(0, HB // L, scan_body, (zeros, zeros))
    cnt1 = jnp.max(cnt1v)
    cnt3 = jnp.max(cnt3v)

    def extract_table(e_t, tl, gsh, hitb, hitv, count, W):
        ngroups = (count + (GRP - 1)) // GRP

        def fill(c, _):
            slot = c * L + iota
            m = slot >= count
            plsc.store_scatter(hitb, [jax.lax.shift_right_logical(slot, 7),
                                      slot & 127], bfill, mask=m)
            return _
        lax.fori_loop(count // L, ngroups * (GRP // L), fill, jnp.int32(0))

        def group(g, _):
            gbase = g * GRP

            def chunk_body(c, off, tail):
                q = 2 * g + jax.lax.shift_right_logical(c, 3)
                qv = jnp.full((L,), q, jnp.int32)
                slot = c * L + iota
                valid = (gbase + slot) < count
                vv = plsc.load_gather(hitv, [qv, slot & 127])
                blk = jax.lax.shift_right_logical(vv, 7)
                if tail:
                    mp = (blk == NFULL) & valid
                    srow = vv - TAIL0
                else:
                    mp = (blk >= off) & (blk < off + PASS_BLKS) & valid
                    srow = (blk - off) * 128 + (vv & 127)

                @pl.when(jnp.any(mp))
                def _():
                    for f in range(W):
                        fv = jnp.full((L,), f, jnp.int32)
                        if tail:
                            x = plsc.load_gather(tl, [srow, fv], mask=mp)
                        else:
                            x = plsc.load_gather(sblk, [fv, srow], mask=mp)
                        plsc.store_scatter(stage, [slot, fv], x, mask=mp)

            for p in range(NPASS):
                off = jnp.minimum(lo + PASS_BLKS * p, MAXOFF)
                offc = pl.multiple_of(off * 128, 128)
                pltpu.sync_copy(e_t.at[:, pl.ds(offc, PASS_BLKS * 128)],
                                sblk.at[pl.ds(0, W), :])

                def pass_chunk(c, _c, _off=off):
                    chunk_body(c, _off, False)
                    return _c
                lax.fori_loop(0, GRP // L, pass_chunk, jnp.int32(0))

            def tail_chunk(c, _c):
                chunk_body(c, jnp.int32(0), True)
                return _c
            lax.fori_loop(0, GRP // L, tail_chunk, jnp.int32(0))

            pltpu.async_copy(stage.at[pl.ds(0, 128), :],
                             gsh.at[hitb.at[2 * g]], sem).wait()
            pltpu.async_copy(stage.at[pl.ds(128, 128), :],
                             gsh.at[hitb.at[2 * g + 1]], sem).wait()
            return _

        lax.fori_loop(0, ngroups, group, jnp.int32(0))

    extract_table(e1t, tl1, gsh1, hitb1, hitv1, cnt1, W1)
    extract_table(e3t, tl3, gsh3, hitb3, hitv3, cnt3, W3)

    plsc.subcore_barrier()
    tb = t * (HB // _NS)
    src_rows = pl.ds(pl.multiple_of(tb, 128), HB // _NS)
    dst_rows = pl.ds(pl.multiple_of(bbase + tb, 128), HB // _NS)
    pltpu.sync_copy(gsh1.at[src_rows, :], g1.at[dst_rows, :])
    pltpu.sync_copy(gsh3.at[src_rows, :], g3.at[dst_rows, :])


@functools.partial(
    pl.kernel,
    mesh=_mesh,
    out_type=(
        jax.ShapeDtypeStruct((H0W, B), jnp.float32),
        jax.ShapeDtypeStruct((H1W, B), jnp.float32),
    ),
    scratch_types=[
        pltpu.VMEM((2, BPW), jnp.int32),
        pltpu.VMEM((1, BPW), jnp.int32),
        pltpu.VMEM((D0N, BPW), jnp.float32),
        pltpu.VMEM((D1N, BPW), jnp.float32),
        pltpu.VMEM((W2, V2), jnp.float32),
        pltpu.VMEM((BPW, 128), jnp.float32),
        pltpu.VMEM((BPW, 128), jnp.float32),
        pltpu.VMEM((H0W, BPW), jnp.float32),
        pltpu.VMEM((H1W, BPW), jnp.float32),
    ],
    compiler_params=_params,
)
def _sc_concat(x0ct, x1ct, x0nt, x1nt, e2t, g1, g3, h0t, h1t,
               cv0, cv1, nb0, nb1, e2v, g1v, g3v, cbuf0, cbuf1):
    wid = lax.axis_index("s") * _NC + lax.axis_index("c")
    base = wid * BPW
    cols = pl.ds(pl.multiple_of(base, 128), BPW)
    iota = lax.iota(jnp.int32, L)
    ones = jnp.full((L,), 1, jnp.int32)

    pltpu.sync_copy(x0ct.at[:, cols], cv0)
    pltpu.sync_copy(x1ct.at[:, cols], cv1)
    pltpu.sync_copy(x0nt.at[:, cols], nb0)
    pltpu.sync_copy(x1nt.at[:, cols], nb1)
    pltpu.sync_copy(e2t, e2v)
    pltpu.sync_copy(g1.at[pl.ds(pl.multiple_of(base, 128), BPW), :], g1v)
    pltpu.sync_copy(g3.at[pl.ds(pl.multiple_of(base, 128), BPW), :], g3v)

    def chunk(k, _):
        it = iota + k * L

        for src, dst, nrows, r0 in ((nb0, cbuf0, D0N, 0),
                                    (nb1, cbuf1, D1N, 0)):
            for f in range(nrows):
                fv = jnp.full((L,), f, jnp.int32)
                x = plsc.load_gather(src, [fv, it])
                plsc.store_scatter(dst, [fv + r0, it], x)

        for src, dst, nrows, r0 in ((g1v, cbuf0, W1, D0N),
                                    (g3v, cbuf1, W3, D1N)):
            for f in range(nrows):
                fv = jnp.full((L,), f, jnp.int32)
                x = plsc.load_gather(src, [it, fv])
                plsc.store_scatter(dst, [fv + r0, it], x)

        v2 = plsc.load_gather(cv0, [ones, it])
        for f in range(W2):
            fv = jnp.full((L,), f, jnp.int32)
            x = plsc.load_gather(e2v, [fv, v2])
            plsc.store_scatter(cbuf0, [fv + (D0N + W1), it], x)
        return _

    lax.fori_loop(0, BPW // L, chunk, jnp.int32(0))

    pltpu.sync_copy(cbuf0, h0t.at[:, cols])
    pltpu.sync_copy(cbuf1, h1t.at[:, cols])


def kernel(X0_num, X0_cat, X1_num, X1_cat, X2_num, E_cat1, E_cat2, E_cat3):
    E1t, E3t, E2t = E_cat1.T, E_cat3.T, E_cat2.T
    X0ct, X1ct = X0_cat.T, X1_cat.T
    X0nt, X1nt = X0_num.T, X1_num.T
    E1tl = E_cat1[TAIL0:, :]
    E3tl = E_cat3[TAIL0:, :]
    G1, G3 = _sc_gather(X0ct, X1ct, E1t, E3t, E1tl, E3tl)
    H0t, H1t = _sc_concat(X0ct, X1ct, X0nt, X1nt, E2t, G1, G3)
    return (H0t.T, H1t.T, X2_num)


# speedup vs baseline: 2.8967x; 1.2540x over previous
import functools
import jax
import jax.numpy as jnp
from jax import lax
from jax.experimental import pallas as pl
from jax.experimental.pallas import tpu as pltpu
from jax.experimental.pallas import tpu_sc as plsc

B = 4096
D0N, D1N = 10, 20
W1, W2, W3 = 32, 16, 64
H0W = D0N + W1 + W2
H1W = D1N + W3

_info = plsc.get_sparse_core_info()
_NC, _NS = _info.num_cores, _info.num_subcores
NW = _NC * _NS
BPW = B // NW
L = 16


@functools.partial(
    pl.kernel,
    mesh=plsc.VectorSubcoreMesh(core_axis_name="c", subcore_axis_name="s"),
    out_type=(
        jax.ShapeDtypeStruct((B, H0W), jnp.float32),
        jax.ShapeDtypeStruct((B, H1W), jnp.float32),
    ),
    scratch_types=[
        pltpu.VMEM((BPW, 2), jnp.int32),
        pltpu.VMEM((BPW, 1), jnp.int32),
        pltpu.VMEM((BPW,), jnp.int32),
        pltpu.VMEM((BPW,), jnp.int32),
        pltpu.VMEM((BPW,), jnp.int32),
        pltpu.VMEM((BPW, D0N), jnp.float32),
        pltpu.VMEM((BPW, D1N), jnp.float32),
        pltpu.VMEM((BPW, W1), jnp.float32),
        pltpu.VMEM((BPW, W2), jnp.float32),
        pltpu.VMEM((BPW, W3), jnp.float32),
        pltpu.VMEM((BPW, H0W), jnp.float32),
        pltpu.VMEM((BPW, H1W), jnp.float32),
        pltpu.SemaphoreType.DMA,
        pltpu.SemaphoreType.DMA,
    ],
    compiler_params=pltpu.CompilerParams(
        use_tc_tiling_on_sc=False, needs_layout_passes=False),
)
def _embed_concat(x0c, x1c, x0n, x1n, e1, e2, e3, h0, h1,
                  c01_v, c1m_v, i0a_v, i0b_v, i1_v, n0_v, n1_v,
                  g0a_v, g0b_v, g1_v, buf0, buf1, sem0, sem1):
    wid = lax.axis_index("s") * _NC + lax.axis_index("c")
    base = wid * BPW
    rows = pl.ds(base, BPW)
    pltpu.sync_copy(x0c.at[rows, :], c01_v)
    pltpu.sync_copy(x1c.at[rows, :], c1m_v)

    iota = lax.iota(jnp.int32, L)
    zeros = jnp.zeros((L,), jnp.int32)
    ones = jnp.full((L,), 1, jnp.int32)
    m10 = iota < D0N

    for k in range(BPW // L):
        ridx = iota + (k * L)
        sl = pl.ds(k * L, L)
        i0a_v[sl] = plsc.load_gather(c01_v, [ridx, zeros])
        i0b_v[sl] = plsc.load_gather(c01_v, [ridx, ones])
        i1_v[sl] = plsc.load_gather(c1m_v, [ridx, zeros])

    cp_n0 = pltpu.async_copy(x0n.at[rows, :], n0_v, sem0)
    g0a = pltpu.async_copy(e1.at[i0a_v], g0a_v, sem0)
    g0b = pltpu.async_copy(e2.at[i0b_v], g0b_v, sem0)
    cp_n1 = pltpu.async_copy(x1n.at[rows, :], n1_v, sem1)
    g1 = pltpu.async_copy(e3.at[i1_v], g1_v, sem1)

    cp_n0.wait()
    g0a.wait()
    g0b.wait()

    def body0(r, _):
        rsp = jnp.full((L,), r, jnp.int32)
        x = plsc.load_gather(n0_v, [rsp, iota], mask=m10)
        plsc.store_scatter(buf0, [rsp, iota], x, mask=m10)
        for c in (0, 16):
            x = g0a_v[r, pl.ds(c, L)]
            plsc.store_scatter(buf0, [rsp, iota + (D0N + c)], x)
        x = g0b_v[r, pl.ds(0, L)]
        plsc.store_scatter(buf0, [rsp, iota + (D0N + W1)], x)
        return _

    lax.fori_loop(0, BPW, body0, None)
    out0 = pltpu.async_copy(buf0, h0.at[rows], sem0)

    cp_n1.wait()
    g1.wait()

    def body1(r, _):
        rsp = jnp.full((L,), r, jnp.int32)
        for c in (0, D1N - L):
            x = plsc.load_gather(n1_v, [rsp, iota + c])
            plsc.store_scatter(buf1, [rsp, iota + c], x)
        for c in (0, 16, 32, 48):
            x = g1_v[r, pl.ds(c, L)]
            plsc.store_scatter(buf1, [rsp, iota + (D1N + c)], x)
        return _

    lax.fori_loop(0, BPW, body1, None)
    out0.wait()
    pltpu.sync_copy(buf1, h1.at[rows])


def kernel(X0_num, X0_cat, X1_num, X1_cat, X2_num, E_cat1, E_cat2, E_cat3):
    H0, H1 = _embed_concat(X0_cat, X1_cat, X0_num, X1_num,
                           E_cat1, E_cat2, E_cat3)
    return (H0, H1, X2_num)
